# baseline (speedup 1.0000x reference)
import functools

import jax
import jax.numpy as jnp
from jax import lax
from jax.experimental import pallas as pl
from jax.experimental.pallas import tpu as pltpu
from jax.experimental.pallas import tpu_sc as plsc


def kernel(inputs, pos_embedding):
    B, S = inputs.shape
    D = pos_embedding.shape[1]

    B_TC = B // 2
    B_SC = B - B_TC

    mesh = plsc.VectorSubcoreMesh(core_axis_name="c", subcore_axis_name="s")
    NC, NS = mesh.num_cores, mesh.num_subcores
    NW = NC * NS
    rows_w = S // NW
    R = min(rows_w, 64)
    n_chunks = rows_w // R

    @functools.partial(
        pl.kernel,
        mesh=mesh,
        out_type=jax.ShapeDtypeStruct((B_SC * S, D), jnp.float32),
        scratch_types=[
            pltpu.VMEM((R, D), jnp.float32),
            pltpu.SemaphoreType.DMA,
        ],
    )
    def sc_broadcast(table_hbm, out_hbm, buf, sem):
        wid = lax.axis_index("s") * NC + lax.axis_index("c")
        base = wid * rows_w
        for c in range(n_chunks):
            off = base + c * R
            pltpu.async_copy(table_hbm.at[pl.ds(off, R)], buf, sem).wait()
            for b in range(B_SC):
                pltpu.sync_copy(buf, out_hbm.at[pl.ds(b * S + off, R)])

    BS = 512

    def tc_body(table_ref, out_ref):
        out_ref[...] = jnp.broadcast_to(table_ref[...][None], out_ref.shape)

    tc_broadcast = pl.pallas_call(
        tc_body,
        grid=(S // BS,),
        in_specs=[pl.BlockSpec((BS, D), lambda i: (i, 0))],
        out_specs=pl.BlockSpec((B_TC, BS, D), lambda i: (0, i, 0)),
        out_shape=jax.ShapeDtypeStruct((B_TC, S, D), jnp.float32),
    )

    sc_part = sc_broadcast(pos_embedding).reshape(B_SC, S, D)
    tc_part = tc_broadcast(pos_embedding)
    return jnp.concatenate([tc_part, sc_part], axis=0)


# speedup vs baseline: 2.2985x; 2.2985x over previous
import functools

import jax
import jax.numpy as jnp
from jax import lax
from jax.experimental import pallas as pl
from jax.experimental.pallas import tpu as pltpu
from jax.experimental.pallas import tpu_sc as plsc


def kernel(inputs, pos_embedding):
    B, S = inputs.shape
    D = pos_embedding.shape[1]

    mesh = plsc.VectorSubcoreMesh(core_axis_name="c", subcore_axis_name="s")
    NC, NS = mesh.num_cores, mesh.num_subcores
    NW = NC * NS
    rows_w = S // NW
    R = min(rows_w, 64)
    n_chunks = rows_w // R

    @functools.partial(
        pl.kernel,
        mesh=mesh,
        out_type=jax.ShapeDtypeStruct((B * S, D), jnp.float32),
        scratch_types=[
            pltpu.VMEM((R, D), jnp.float32),
            pltpu.SemaphoreType.DMA,
            pltpu.SemaphoreType.DMA,
        ],
    )
    def sc_broadcast(table_hbm, out_hbm, buf, rsem, wsem):
        wid = lax.axis_index("s") * NC + lax.axis_index("c")
        base = wid * rows_w
        for c in range(n_chunks):
            off = base + c * R
            pltpu.async_copy(table_hbm.at[pl.ds(off, R)], buf, rsem).wait()
            writes = [
                pltpu.async_copy(
                    buf, out_hbm.at[pl.ds(b * S + off, R)], wsem)
                for b in range(B)
            ]
            for w in writes:
                w.wait()

    return sc_broadcast(pos_embedding).reshape(B, S, D)
